# baseline (speedup 1.0000x reference)
import functools

import numpy as np
import jax
import jax.numpy as jnp
from jax import lax
from jax.experimental import pallas as pl
from jax.experimental.pallas import tpu as pltpu
from jax.experimental.pallas import tpu_sc as plsc

N_ROWS = 128
N_CLASSES = 100000
NUM_BINS = 100
ROWS_PER_BLOCK = 32
N_BLOCKS = N_ROWS // ROWS_PER_BLOCK
EPS = float(np.finfo(np.float32).eps)


def _sc_gather_kernel(flat_logits, flat_idx, xt_out, idx_v, xt_v, sem):
    cid = lax.axis_index("c")
    sid = lax.axis_index("s")

    @pl.when((cid == 0) & (sid == 0))
    def _():
        pltpu.sync_copy(flat_idx, idx_v)
        pltpu.async_copy(flat_logits.at[idx_v], xt_v, sem).wait()
        pltpu.sync_copy(xt_v, xt_out)


def _stage1_kernel(logits_ref, p_ref, m_ref):
    x = logits_ref[...]
    m = jnp.max(x, axis=1, keepdims=True)
    s = jnp.sum(jnp.exp(x - m), axis=1, keepdims=True)
    p_ref[...] = 1.0 / s
    m_ref[...] = m


def _stage2_kernel(p_ref, m_ref, xt_ref, out_ref):
    pc = p_ref[...]
    hc = (xt_ref[...] == m_ref[...]).astype(jnp.float32)
    row_i = jax.lax.broadcasted_iota(jnp.int32, (N_ROWS, N_ROWS), 0)
    col_j = jax.lax.broadcasted_iota(jnp.int32, (N_ROWS, N_ROWS), 1)
    eye = (row_i == col_j).astype(jnp.float32)
    p_row = jax.lax.dot_general(pc, eye, (((0,), (0,)), ((), ())),
                                preferred_element_type=jnp.float32)
    less = p_row < pc
    tie = (p_row == pc) & (col_j < row_i)
    rank = jnp.sum((less | tie).astype(jnp.int32), axis=1, keepdims=True)
    lane_b = jax.lax.broadcasted_iota(jnp.int32, (1, NUM_BINS + 1), 1)
    e_raw = jnp.minimum(
        jnp.round(lane_b.astype(jnp.float32) * (N_ROWS / NUM_BINS)),
        float(N_ROWS - 1)).astype(jnp.int32)
    e_idx = jnp.where(lane_b < NUM_BINS, e_raw, -1)
    onehot_e = (rank == e_idx).astype(jnp.float32)
    ub = (jnp.sum(pc * onehot_e, axis=0, keepdims=True)
          + jnp.where(lane_b == NUM_BINS, 1.0, 0.0))
    bin_idx = jnp.sum((ub <= pc).astype(jnp.int32), axis=1, keepdims=True)
    b_iota = jax.lax.broadcasted_iota(jnp.int32, (N_ROWS, NUM_BINS + 2), 1)
    onehot_b = (bin_idx == b_iota).astype(jnp.float32)
    counts = jnp.sum(onehot_b, axis=0, keepdims=True) + EPS
    sums = jnp.sum(pc * onehot_b, axis=0, keepdims=True)
    hits = jnp.sum(hc * onehot_b, axis=0, keepdims=True)
    err = jnp.square(hits / counts - sums / counts)
    ce = jnp.sum(jnp.abs(counts * (1.0 / N_ROWS) * err),
                 axis=1, keepdims=True)
    out_ref[...] = jnp.sqrt(ce)


@functools.partial(
    pl.kernel,
    mesh=plsc.VectorSubcoreMesh(core_axis_name="c", subcore_axis_name="s"),
    out_type=jax.ShapeDtypeStruct((N_ROWS,), jnp.float32),
    scratch_types=[
        pltpu.VMEM((N_ROWS,), jnp.int32),
        pltpu.VMEM((N_ROWS,), jnp.float32),
        pltpu.SemaphoreType.DMA,
    ],
)
def _sc_gather(flat_logits, flat_idx, xt_out, idx_v, xt_v, sem):
    _sc_gather_kernel(flat_logits, flat_idx, xt_out, idx_v, xt_v, sem)


def kernel(logits, targets):
    flat_idx = (jnp.arange(N_ROWS, dtype=jnp.int32) * N_CLASSES
                + targets.astype(jnp.int32))
    xt = _sc_gather(logits.reshape(-1), flat_idx)
    p, m = pl.pallas_call(
        _stage1_kernel,
        grid=(N_BLOCKS,),
        in_specs=[
            pl.BlockSpec((ROWS_PER_BLOCK, N_CLASSES), lambda i: (i, 0)),
        ],
        out_specs=[
            pl.BlockSpec((ROWS_PER_BLOCK, 1), lambda i: (i, 0)),
            pl.BlockSpec((ROWS_PER_BLOCK, 1), lambda i: (i, 0)),
        ],
        out_shape=[
            jax.ShapeDtypeStruct((N_ROWS, 1), jnp.float32),
            jax.ShapeDtypeStruct((N_ROWS, 1), jnp.float32),
        ],
        compiler_params=pltpu.CompilerParams(
            dimension_semantics=("parallel",)),
    )(logits)
    out = pl.pallas_call(
        _stage2_kernel,
        out_shape=jax.ShapeDtypeStruct((1, 1), jnp.float32),
    )(p, m, xt.reshape(N_ROWS, 1))
    return out.reshape(())


# speedup vs baseline: 2.2916x; 2.2916x over previous
import numpy as np
import jax
import jax.numpy as jnp
from jax.experimental import pallas as pl
from jax.experimental.pallas import tpu as pltpu

N_ROWS = 128
N_CLASSES = 100000
NUM_BINS = 100
ROWS_PER_BLOCK = 32
N_BLOCKS = N_ROWS // ROWS_PER_BLOCK
EPS = float(np.finfo(np.float32).eps)


def _gce_kernel(logits_ref, tgt_tile_ref, tgt_off_ref, out_ref, p_col, h_col):
    i = pl.program_id(0)
    x = logits_ref[...]
    m = jnp.max(x, axis=1, keepdims=True)
    s = jnp.sum(jnp.exp(x - m), axis=1, keepdims=True)
    p = 1.0 / s
    lane = jax.lax.broadcasted_iota(jnp.int32, (1, 128), 1)
    xt_rows = []
    for k in range(ROWS_PER_BLOCK):
        win = logits_ref[pl.ds(k, 1), pl.ds(tgt_tile_ref[k, 0] * 128, 128)]
        xt_rows.append(jnp.max(
            jnp.where(lane == tgt_off_ref[k, 0], win, -jnp.inf),
            axis=1, keepdims=True))
    xt = jnp.concatenate(xt_rows, axis=0)
    hit = (xt == m).astype(jnp.float32)
    p_col[pl.ds(i * ROWS_PER_BLOCK, ROWS_PER_BLOCK), :] = p
    h_col[pl.ds(i * ROWS_PER_BLOCK, ROWS_PER_BLOCK), :] = hit

    @pl.when(i == N_BLOCKS - 1)
    def _tail():
        pc = p_col[...]
        hc = h_col[...]
        row_i = jax.lax.broadcasted_iota(jnp.int32, (N_ROWS, N_ROWS), 0)
        col_j = jax.lax.broadcasted_iota(jnp.int32, (N_ROWS, N_ROWS), 1)
        eye = (row_i == col_j).astype(jnp.float32)
        p_row = jax.lax.dot_general(pc, eye, (((0,), (0,)), ((), ())),
                                    preferred_element_type=jnp.float32)
        less = p_row < pc
        tie = (p_row == pc) & (col_j < row_i)
        rank = jnp.sum((less | tie).astype(jnp.int32), axis=1, keepdims=True)
        lane_b = jax.lax.broadcasted_iota(jnp.int32, (1, NUM_BINS + 1), 1)
        e_raw = jnp.minimum(
            jnp.round(lane_b.astype(jnp.float32) * (N_ROWS / NUM_BINS)),
            float(N_ROWS - 1)).astype(jnp.int32)
        e_idx = jnp.where(lane_b < NUM_BINS, e_raw, -1)
        onehot_e = (rank == e_idx).astype(jnp.float32)
        ub = (jnp.sum(pc * onehot_e, axis=0, keepdims=True)
              + jnp.where(lane_b == NUM_BINS, 1.0, 0.0))
        bin_idx = jnp.sum((ub <= pc).astype(jnp.int32), axis=1, keepdims=True)
        b_iota = jax.lax.broadcasted_iota(jnp.int32, (N_ROWS, NUM_BINS + 2), 1)
        onehot_b = (bin_idx == b_iota).astype(jnp.float32)
        counts = jnp.sum(onehot_b, axis=0, keepdims=True) + EPS
        sums = jnp.sum(pc * onehot_b, axis=0, keepdims=True)
        hits = jnp.sum(hc * onehot_b, axis=0, keepdims=True)
        err = jnp.square(hits / counts - sums / counts)
        ce = jnp.sum(jnp.abs(counts * (1.0 / N_ROWS) * err),
                     axis=1, keepdims=True)
        out_ref[...] = jnp.sqrt(ce)


def kernel(logits, targets):
    out = pl.pallas_call(
        _gce_kernel,
        grid=(N_BLOCKS,),
        in_specs=[
            pl.BlockSpec((ROWS_PER_BLOCK, N_CLASSES), lambda i: (i, 0)),
            pl.BlockSpec((ROWS_PER_BLOCK, 1), lambda i: (i, 0),
                         memory_space=pltpu.SMEM),
            pl.BlockSpec((ROWS_PER_BLOCK, 1), lambda i: (i, 0),
                         memory_space=pltpu.SMEM),
        ],
        out_specs=pl.BlockSpec((1, 1), lambda i: (0, 0)),
        out_shape=jax.ShapeDtypeStruct((1, 1), jnp.float32),
        scratch_shapes=[
            pltpu.VMEM((N_ROWS, 1), jnp.float32),
            pltpu.VMEM((N_ROWS, 1), jnp.float32),
        ],
    )(logits,
      (targets // 128).reshape(N_ROWS, 1),
      (targets % 128).reshape(N_ROWS, 1))
    return out.reshape(())
